# baseline (speedup 1.0000x reference)
import functools

import jax
import jax.numpy as jnp
from jax import lax
from jax.experimental import pallas as pl
from jax.experimental.pallas import tpu as pltpu
from jax.experimental.pallas import tpu_sc as plsc

_N_PRED = 20
_M = 12


def _soft_sqrt(x):
    i = plsc.bitcast(x, jnp.int32)
    y = plsc.bitcast(jnp.int32(0x5F3759DF) - lax.shift_right_logical(i, 1),
                     jnp.float32)
    xh = 0.5 * x
    y = y * (1.5 - xh * y * y)
    y = y * (1.5 - xh * y * y)
    y = y * (1.5 - xh * y * y)
    return x * y


def _soft_log(x):
    i = plsc.bitcast(x, jnp.int32)
    e = lax.shift_right_logical(i, 23) - 127
    m = plsc.bitcast(
        jnp.bitwise_or(jnp.bitwise_and(i, 0x007FFFFF), 0x3F800000),
        jnp.float32)
    big = m > 1.4142135381698608
    m = jnp.where(big, 0.5 * m, m)
    ef = e.astype(jnp.float32) + jnp.where(big, 1.0, 0.0)
    t = (m - 1.0) / (m + 1.0)
    t2 = t * t
    p = 1.0 + t2 * (0.3333333333 + t2 * (0.2 + t2 * (0.1428571429
                                                     + t2 * 0.1111111111)))
    return ef * 0.6931471805599453 + (2.0 * t) * p


@functools.partial(
    pl.kernel,
    out_type=jax.ShapeDtypeStruct((16,), jnp.float32),
    mesh=plsc.VectorSubcoreMesh(core_axis_name="c", subcore_axis_name="s"),
    compiler_params=pltpu.CompilerParams(needs_layout_passes=False),
    scratch_types=[
        pltpu.VMEM((144,), jnp.float32),
        pltpu.VMEM((64,), jnp.float32),
        pltpu.VMEM((32,), jnp.float32),
        pltpu.VMEM((16,), jnp.float32),
    ],
)
def _sc_loss(pred_hbm, label_hbm, out_hbm, pred_v, label_v, mask_v, out_v):
    cid = lax.axis_index("c")
    sid = lax.axis_index("s")
    if True:
        pltpu.sync_copy(pred_hbm, pred_v.at[pl.ds(8, 4 * _N_PRED)])
        pltpu.sync_copy(label_hbm, label_v.at[pl.ds(0, 4 * _M)])
        lanes = lax.iota(jnp.int32, 16)
        l4 = 4 * lanes
        lx = plsc.load_gather(label_v, [l4])
        ly = plsc.load_gather(label_v, [l4 + 1])
        lr = plsc.load_gather(label_v, [l4 + 2])
        bc = jnp.full((16,), 1e30, jnp.float32)
        bi = jnp.zeros((16,), jnp.int32)
        for n in range(_N_PRED):
            px = plsc.load_gather(pred_v, [jnp.full((16,), 8 + 4 * n, jnp.int32)])
            py = plsc.load_gather(pred_v,
                                  [jnp.full((16,), 8 + 4 * n + 1, jnp.int32)])
            pr = plsc.load_gather(pred_v,
                                  [jnp.full((16,), 8 + 4 * n + 2, jnp.int32)])
            dx = lx - px
            dy = ly - py
            c = 0.5 * _soft_sqrt(dx * dx + dy * dy) + 0.5 * jnp.abs(lr - pr)
            upd = c < bc
            bc = jnp.where(upd, c, bc)
            bi = jnp.where(upd, jnp.full((16,), n, jnp.int32), bi)
        valid = lanes < _M
        b4 = 4 * bi + 8
        sx = plsc.load_gather(pred_v, [b4])
        sy = plsc.load_gather(pred_v, [b4 + 1])
        sr = plsc.load_gather(pred_v, [b4 + 2])
        sp = plsc.load_gather(pred_v, [b4 + 3])
        ddx = lx - sx
        ddy = ly - sy
        pairs = (0.5 * _soft_sqrt(ddx * ddx + ddy * ddy)
                 + 0.5 * jnp.abs(lr - sr) - _soft_log(sp + 1e-6))
        pair_sum = jnp.sum(jnp.where(valid, pairs, 0.0))
        mask_v[pl.ds(0, 16)] = jnp.zeros((16,), jnp.float32)
        mask_v[pl.ds(16, 16)] = jnp.zeros((16,), jnp.float32)
        plsc.store_scatter(mask_v, [bi], jnp.ones((16,), jnp.float32),
                           mask=valid)
        r0 = plsc.load_gather(pred_v, [l4 + 10])
        r1 = plsc.load_gather(pred_v, [l4 + 74])
        p0 = plsc.load_gather(pred_v, [l4 + 11])
        p1 = plsc.load_gather(pred_v, [l4 + 75])
        m0 = mask_v[pl.ds(0, 16)]
        m1 = mask_v[pl.ds(16, 16)]
        u0 = (-_soft_log(1.0 - p0 + 1e-6) + 0.5 * r0) * 0.5
        u1 = (-_soft_log(1.0 - p1 + 1e-6) + 0.5 * r1) * 0.5
        un = (jnp.where(m0 == 0.0, u0, 0.0)
              + jnp.where(jnp.logical_and(m1 == 0.0, lanes < _N_PRED - 16),
                          u1, 0.0))
        loss = pair_sum * (1.0 / _M) + jnp.sum(un) * (1.0 / (_N_PRED - _M))
        out_v[...] = jnp.full((16,), loss)

        @pl.when(jnp.logical_and(cid == 0, sid == 0))
        def _():
            pltpu.sync_copy(out_v, out_hbm)


def kernel(pred, label):
    return _sc_loss(pred.reshape(4 * _N_PRED), label.reshape(4 * _M))[0]


# speedup vs baseline: 1.0802x; 1.0802x over previous
import functools

import jax
import jax.numpy as jnp
from jax import lax
from jax.experimental import pallas as pl
from jax.experimental.pallas import tpu as pltpu
from jax.experimental.pallas import tpu_sc as plsc

_N_PRED = 20
_M = 12


def _soft_sqrt(x):
    i = plsc.bitcast(x, jnp.int32)
    y = plsc.bitcast(jnp.int32(0x5F3759DF) - lax.shift_right_logical(i, 1),
                     jnp.float32)
    xh = 0.5 * x
    y = y * (1.5 - xh * y * y)
    y = y * (1.5 - xh * y * y)
    y = y * (1.5 - xh * y * y)
    return x * y


def _soft_log(x):
    i = plsc.bitcast(x, jnp.int32)
    e = lax.shift_right_logical(i, 23) - 127
    m = plsc.bitcast(
        jnp.bitwise_or(jnp.bitwise_and(i, 0x007FFFFF), 0x3F800000),
        jnp.float32)
    big = m > 1.4142135381698608
    m = jnp.where(big, 0.5 * m, m)
    ef = e.astype(jnp.float32) + jnp.where(big, 1.0, 0.0)
    t = (m - 1.0) / (m + 1.0)
    t2 = t * t
    p = 1.0 + t2 * (0.3333333333 + t2 * (0.2 + t2 * (0.1428571429
                                                     + t2 * 0.1111111111)))
    return ef * 0.6931471805599453 + (2.0 * t) * p


@functools.partial(
    pl.kernel,
    out_type=jax.ShapeDtypeStruct((16,), jnp.float32),
    mesh=plsc.VectorSubcoreMesh(core_axis_name="c", subcore_axis_name="s",
                               num_cores=1),
    compiler_params=pltpu.CompilerParams(needs_layout_passes=False),
    scratch_types=[
        pltpu.VMEM((144,), jnp.float32),
        pltpu.VMEM((64,), jnp.float32),
        pltpu.VMEM((32,), jnp.float32),
        pltpu.VMEM((16,), jnp.float32),
    ],
)
def _sc_loss(pred_hbm, label_hbm, out_hbm, pred_v, label_v, mask_v, out_v):
    cid = lax.axis_index("c")
    sid = lax.axis_index("s")

    @pl.when(jnp.logical_and(cid == 0, sid == 0))
    def _():
        pltpu.sync_copy(pred_hbm, pred_v.at[pl.ds(8, 4 * _N_PRED)])
        pltpu.sync_copy(label_hbm, label_v.at[pl.ds(0, 4 * _M)])
        lanes = lax.iota(jnp.int32, 16)
        l4 = 4 * lanes
        lx = plsc.load_gather(label_v, [l4])
        ly = plsc.load_gather(label_v, [l4 + 1])
        lr = plsc.load_gather(label_v, [l4 + 2])
        bc = jnp.full((16,), 1e30, jnp.float32)
        bi = jnp.zeros((16,), jnp.int32)
        for n in range(_N_PRED):
            px = plsc.load_gather(pred_v, [jnp.full((16,), 8 + 4 * n, jnp.int32)])
            py = plsc.load_gather(pred_v,
                                  [jnp.full((16,), 8 + 4 * n + 1, jnp.int32)])
            pr = plsc.load_gather(pred_v,
                                  [jnp.full((16,), 8 + 4 * n + 2, jnp.int32)])
            dx = lx - px
            dy = ly - py
            c = 0.5 * _soft_sqrt(dx * dx + dy * dy) + 0.5 * jnp.abs(lr - pr)
            upd = c < bc
            bc = jnp.where(upd, c, bc)
            bi = jnp.where(upd, jnp.full((16,), n, jnp.int32), bi)
        valid = lanes < _M
        b4 = 4 * bi + 8
        sx = plsc.load_gather(pred_v, [b4])
        sy = plsc.load_gather(pred_v, [b4 + 1])
        sr = plsc.load_gather(pred_v, [b4 + 2])
        sp = plsc.load_gather(pred_v, [b4 + 3])
        ddx = lx - sx
        ddy = ly - sy
        pairs = (0.5 * _soft_sqrt(ddx * ddx + ddy * ddy)
                 + 0.5 * jnp.abs(lr - sr) - _soft_log(sp + 1e-6))
        pair_sum = jnp.sum(jnp.where(valid, pairs, 0.0))
        mask_v[pl.ds(0, 16)] = jnp.zeros((16,), jnp.float32)
        mask_v[pl.ds(16, 16)] = jnp.zeros((16,), jnp.float32)
        plsc.store_scatter(mask_v, [bi], jnp.ones((16,), jnp.float32),
                           mask=valid)
        r0 = plsc.load_gather(pred_v, [l4 + 10])
        r1 = plsc.load_gather(pred_v, [l4 + 74])
        p0 = plsc.load_gather(pred_v, [l4 + 11])
        p1 = plsc.load_gather(pred_v, [l4 + 75])
        m0 = mask_v[pl.ds(0, 16)]
        m1 = mask_v[pl.ds(16, 16)]
        u0 = (-_soft_log(1.0 - p0 + 1e-6) + 0.5 * r0) * 0.5
        u1 = (-_soft_log(1.0 - p1 + 1e-6) + 0.5 * r1) * 0.5
        un = (jnp.where(m0 == 0.0, u0, 0.0)
              + jnp.where(jnp.logical_and(m1 == 0.0, lanes < _N_PRED - 16),
                          u1, 0.0))
        loss = pair_sum * (1.0 / _M) + jnp.sum(un) * (1.0 / (_N_PRED - _M))
        out_v[...] = jnp.full((16,), loss)
        pltpu.sync_copy(out_v, out_hbm)


def kernel(pred, label):
    return _sc_loss(pred.reshape(4 * _N_PRED), label.reshape(4 * _M))[0]


# speedup vs baseline: 1.1070x; 1.0248x over previous
import functools

import jax
import jax.numpy as jnp
from jax import lax
from jax.experimental import pallas as pl
from jax.experimental.pallas import tpu as pltpu
from jax.experimental.pallas import tpu_sc as plsc

_N_PRED = 20
_M = 12


def _soft_sqrt(x):
    i = plsc.bitcast(x, jnp.int32)
    y = plsc.bitcast(jnp.int32(0x5F3759DF) - lax.shift_right_logical(i, 1),
                     jnp.float32)
    xh = 0.5 * x
    y = y * (1.5 - xh * y * y)
    y = y * (1.5 - xh * y * y)
    y = y * (1.5 - xh * y * y)
    return x * y


def _soft_log(x):
    i = plsc.bitcast(x, jnp.int32)
    e = lax.shift_right_logical(i, 23) - 127
    m = plsc.bitcast(
        jnp.bitwise_or(jnp.bitwise_and(i, 0x007FFFFF), 0x3F800000),
        jnp.float32)
    big = m > 1.4142135381698608
    m = jnp.where(big, 0.5 * m, m)
    ef = e.astype(jnp.float32) + jnp.where(big, 1.0, 0.0)
    t = (m - 1.0) / (m + 1.0)
    t2 = t * t
    p = 1.0 + t2 * (0.3333333333 + t2 * (0.2 + t2 * (0.1428571429
                                                     + t2 * 0.1111111111)))
    return ef * 0.6931471805599453 + (2.0 * t) * p


@functools.partial(
    pl.kernel,
    out_type=jax.ShapeDtypeStruct((16,), jnp.float32),
    mesh=plsc.VectorSubcoreMesh(core_axis_name="c", subcore_axis_name="s",
                               num_cores=1),
    compiler_params=pltpu.CompilerParams(needs_layout_passes=False),
    scratch_types=[
        pltpu.VMEM((144,), jnp.float32),
        pltpu.VMEM((64,), jnp.float32),
        pltpu.VMEM((32,), jnp.float32),
        pltpu.VMEM((16,), jnp.float32),
        pltpu.SemaphoreType.DMA,
        pltpu.SemaphoreType.DMA,
    ],
)
def _sc_loss(pred_hbm, label_hbm, out_hbm, pred_v, label_v, mask_v,
             out_v, sem_p, sem_l):
    cid = lax.axis_index("c")
    sid = lax.axis_index("s")

    @pl.when(jnp.logical_and(cid == 0, sid == 0))
    def _():
        cp = pltpu.async_copy(pred_hbm, pred_v.at[pl.ds(8, 4 * _N_PRED)], sem_p)
        cl = pltpu.async_copy(label_hbm, label_v.at[pl.ds(0, 4 * _M)], sem_l)
        cl.wait()
        lanes = lax.iota(jnp.int32, 16)
        l4 = 4 * lanes
        lx = plsc.load_gather(label_v, [l4])
        ly = plsc.load_gather(label_v, [l4 + 1])
        lr = plsc.load_gather(label_v, [l4 + 2])
        cp.wait()
        bc = jnp.full((16,), 1e30, jnp.float32)
        bi = jnp.zeros((16,), jnp.int32)
        for n in range(_N_PRED):
            px = plsc.load_gather(pred_v, [jnp.full((16,), 8 + 4 * n, jnp.int32)])
            py = plsc.load_gather(pred_v,
                                  [jnp.full((16,), 8 + 4 * n + 1, jnp.int32)])
            pr = plsc.load_gather(pred_v,
                                  [jnp.full((16,), 8 + 4 * n + 2, jnp.int32)])
            dx = lx - px
            dy = ly - py
            c = 0.5 * _soft_sqrt(dx * dx + dy * dy) + 0.5 * jnp.abs(lr - pr)
            upd = c < bc
            bc = jnp.where(upd, c, bc)
            bi = jnp.where(upd, jnp.full((16,), n, jnp.int32), bi)
        valid = lanes < _M
        b4 = 4 * bi + 8
        sx = plsc.load_gather(pred_v, [b4])
        sy = plsc.load_gather(pred_v, [b4 + 1])
        sr = plsc.load_gather(pred_v, [b4 + 2])
        sp = plsc.load_gather(pred_v, [b4 + 3])
        ddx = lx - sx
        ddy = ly - sy
        pairs = (0.5 * _soft_sqrt(ddx * ddx + ddy * ddy)
                 + 0.5 * jnp.abs(lr - sr) - _soft_log(sp + 1e-6))
        pair_sum = jnp.sum(jnp.where(valid, pairs, 0.0))
        mask_v[pl.ds(0, 16)] = jnp.zeros((16,), jnp.float32)
        mask_v[pl.ds(16, 16)] = jnp.zeros((16,), jnp.float32)
        plsc.store_scatter(mask_v, [bi], jnp.ones((16,), jnp.float32),
                           mask=valid)
        r0 = plsc.load_gather(pred_v, [l4 + 10])
        r1 = plsc.load_gather(pred_v, [l4 + 74])
        p0 = plsc.load_gather(pred_v, [l4 + 11])
        p1 = plsc.load_gather(pred_v, [l4 + 75])
        m0 = mask_v[pl.ds(0, 16)]
        m1 = mask_v[pl.ds(16, 16)]
        u0 = (-_soft_log(1.0 - p0 + 1e-6) + 0.5 * r0) * 0.5
        u1 = (-_soft_log(1.0 - p1 + 1e-6) + 0.5 * r1) * 0.5
        un = (jnp.where(m0 == 0.0, u0, 0.0)
              + jnp.where(jnp.logical_and(m1 == 0.0, lanes < _N_PRED - 16),
                          u1, 0.0))
        loss = pair_sum * (1.0 / _M) + jnp.sum(un) * (1.0 / (_N_PRED - _M))
        out_v[...] = jnp.full((16,), loss)
        pltpu.sync_copy(out_v, out_hbm)


def kernel(pred, label):
    return _sc_loss(pred.reshape(4 * _N_PRED), label.reshape(4 * _M))[0]
